# baseline (speedup 1.0000x reference)
import jax
import jax.numpy as jnp
from jax.experimental import pallas as pl
from jax.experimental.pallas import tpu as pltpu

_CONTRACT_RHS1 = (((1,), (1,)), ((), ()))


def _sage_kernel(x_ref, a1_ref, a2_ref, a3_ref, wx_ref, wn_ref,
                 bx_ref, bn_ref, g_ref, be_ref, out_ref,
                 mask_s, corr_s, h_s, s_s, s2_s):
    N = a1_ref.shape[0]
    O = wx_ref.shape[0]
    C = out_ref.shape[2]
    B = h_s.shape[0] // N
    n_max = jnp.float32(N - 1)
    inv_n = jnp.float32(1.0) / n_max
    i = pl.program_id(0)

    @pl.when(i == 0)
    def _build_masks():
        row = jax.lax.broadcasted_iota(jnp.int32, (N, N), 0)
        col = jax.lax.broadcasted_iota(jnp.int32, (N, N), 1)
        not_eye = row != col
        ms = [jnp.where((a_ref[...] > 0.0) & not_eye, inv_n, jnp.float32(0.0))
              for a_ref in (a1_ref, a2_ref, a3_ref)]
        m_all = jnp.concatenate(ms, axis=0)
        mask_s[...] = m_all
        corr_s[...] = jnp.float32(1.0) - jnp.sum(m_all, axis=1, keepdims=True)
        s_s[...] = jnp.zeros_like(s_s)
        s2_s[...] = jnp.zeros_like(s2_s)

    @pl.when(i < B)
    def _compute_batch():
        xb = x_ref[0]
        h0 = jax.lax.dot_general(xb, wx_ref[...], _CONTRACT_RHS1,
                                 preferred_element_type=jnp.float32)
        h0 = h0 + bx_ref[...]
        y = jax.lax.dot_general(xb, wn_ref[...], _CONTRACT_RHS1,
                                preferred_element_type=jnp.float32)
        y_last = y[N - 1:N, :]
        agg = jnp.dot(mask_s[...], y, preferred_element_type=jnp.float32)
        agg = agg + corr_s[...] * y_last + bn_ref[...]
        h = jnp.concatenate(
            [h0, agg[0:N, :], agg[N:2 * N, :], agg[2 * N:3 * N, :]], axis=1)
        nrm = jnp.sqrt(jnp.sum(h * h, axis=1, keepdims=True))
        h = h / jnp.maximum(nrm, jnp.float32(1e-12))
        h = jnp.maximum(h, jnp.float32(0.0))
        h_s[pl.ds(i * N, N), :] = h
        s_s[...] = s_s[...] + jnp.sum(h, axis=0, keepdims=True)
        s2_s[...] = s2_s[...] + jnp.sum(h * h, axis=0, keepdims=True)

    @pl.when(i >= B)
    def _write_batch():
        cnt = jnp.float32(B * N)
        mean = s_s[...] / cnt
        var = s2_s[...] / cnt - mean * mean
        scale = g_ref[...] / jnp.sqrt(var + jnp.float32(1e-5))
        shift = be_ref[...] - mean * scale
        h = h_s[pl.ds((i - B) * N, N), :]
        out_ref[0] = h * scale + shift


def kernel(x, adj1, adj2, adj3, Wx_w, Wx_b, Wn_w, Wn_b, bn_gamma, bn_beta):
    B, N, F = x.shape
    O = Wx_w.shape[0]
    C = 4 * O
    const2 = lambda shape: pl.BlockSpec(shape, lambda i: (0, 0))
    out = pl.pallas_call(
        _sage_kernel,
        grid=(2 * B,),
        in_specs=[
            pl.BlockSpec((1, N, F), lambda i: (jnp.minimum(i, B - 1), 0, 0)),
            const2((N, N)), const2((N, N)), const2((N, N)),
            const2((O, F)), const2((O, F)),
            const2((1, O)), const2((1, O)),
            const2((1, C)), const2((1, C)),
        ],
        out_specs=pl.BlockSpec((1, N, C),
                               lambda i: (jnp.maximum(i - B, 0), 0, 0)),
        scratch_shapes=[
            pltpu.VMEM((3 * N, N), jnp.float32),
            pltpu.VMEM((3 * N, 1), jnp.float32),
            pltpu.VMEM((B * N, C), jnp.float32),
            pltpu.VMEM((1, C), jnp.float32),
            pltpu.VMEM((1, C), jnp.float32),
        ],
        out_shape=jax.ShapeDtypeStruct((B, N, C), jnp.float32),
        compiler_params=pltpu.CompilerParams(
            dimension_semantics=("arbitrary",)),
    )(x, adj1, adj2, adj3, Wx_w, Wn_w,
      Wx_b.reshape(1, O), Wn_b.reshape(1, O),
      bn_gamma.reshape(1, C), bn_beta.reshape(1, C))
    return out


# speedup vs baseline: 1.4674x; 1.4674x over previous
import jax
import jax.numpy as jnp
from jax.experimental import pallas as pl

_CONTRACT_RHS1 = (((1,), (1,)), ((), ()))


def _sage_kernel(x_ref, a1_ref, a2_ref, a3_ref, wx_ref, wn_ref,
                 bx_ref, bn_ref, g_ref, be_ref, out_ref):
    B, N, F = x_ref.shape
    C = out_ref.shape[2]
    O = C // 4
    n_max = jnp.float32(N - 1)

    x_all = x_ref[...].reshape(B * N, F)
    h0_all = jax.lax.dot_general(x_all, wx_ref[...], _CONTRACT_RHS1,
                                 preferred_element_type=jnp.float32)
    h0_all = h0_all + bx_ref[...]
    y_all = jax.lax.dot_general(x_all, wn_ref[...], _CONTRACT_RHS1,
                                preferred_element_type=jnp.float32)

    row = jax.lax.broadcasted_iota(jnp.int32, (N, N), 0)
    col = jax.lax.broadcasted_iota(jnp.int32, (N, N), 1)
    not_eye = row != col

    inv_n = jnp.float32(1.0) / n_max
    ms = [jnp.where((a_ref[...] > 0.0) & not_eye, inv_n, jnp.float32(0.0))
          for a_ref in (a1_ref, a2_ref, a3_ref)]
    m_all = jnp.concatenate(ms, axis=0)
    corr = jnp.float32(1.0) - jnp.sum(m_all, axis=1, keepdims=True)

    bn = bn_ref[...]
    s = jnp.zeros((1, C), dtype=jnp.float32)
    s2 = jnp.zeros((1, C), dtype=jnp.float32)
    for b in range(B):
        y_b = y_all[b * N:(b + 1) * N, :]
        y_last = y_all[b * N + N - 1:b * N + N, :]
        agg = jnp.dot(m_all, y_b, preferred_element_type=jnp.float32)
        agg = agg + corr * y_last + bn
        h = jnp.concatenate(
            [h0_all[b * N:(b + 1) * N, :],
             agg[0:N, :], agg[N:2 * N, :], agg[2 * N:3 * N, :]], axis=1)
        nrm = jnp.sqrt(jnp.sum(h * h, axis=1, keepdims=True))
        h = h / jnp.maximum(nrm, jnp.float32(1e-12))
        h = jnp.maximum(h, jnp.float32(0.0))
        out_ref[b] = h
        s = s + jnp.sum(h, axis=0, keepdims=True)
        s2 = s2 + jnp.sum(h * h, axis=0, keepdims=True)

    cnt = jnp.float32(B * N)
    mean = s / cnt
    var = s2 / cnt - mean * mean
    scale = g_ref[...] / jnp.sqrt(var + jnp.float32(1e-5))
    shift = be_ref[...] - mean * scale
    for b in range(B):
        out_ref[b] = out_ref[b] * scale + shift


def kernel(x, adj1, adj2, adj3, Wx_w, Wx_b, Wn_w, Wn_b, bn_gamma, bn_beta):
    B, N, F = x.shape
    O = Wx_w.shape[0]
    C = 4 * O
    out = pl.pallas_call(
        _sage_kernel,
        out_shape=jax.ShapeDtypeStruct((B, N, C), jnp.float32),
    )(x, adj1, adj2, adj3, Wx_w, Wn_w,
      Wx_b.reshape(1, O), Wn_b.reshape(1, O),
      bn_gamma.reshape(1, C), bn_beta.reshape(1, C))
    return out
